# Initial kernel scaffold; baseline (speedup 1.0000x reference)
import functools

import jax
import jax.numpy as jnp
from jax import lax
from jax.experimental import pallas as pl
from jax.experimental.pallas import tpu as pltpu
from jax.experimental.pallas import tpu_sc as plsc

N_NODES = 10000
N_EDGES = 320000
D = 128

NC = 2
NS = 16
NW = NC * NS
EPT = N_EDGES // NW
C = 80
NCHUNK = EPT // C
ROWS_PT = N_NODES // NS

_mesh = plsc.VectorSubcoreMesh(core_axis_name="c", subcore_axis_name="s")


@functools.partial(
    pl.kernel,
    out_type=jax.ShapeDtypeStruct((NC * N_NODES,), jnp.float32),
    mesh=_mesh,
    scratch_types=[
        pltpu.VMEM((C,), jnp.int32),
        pltpu.VMEM((C,), jnp.float32),
        pltpu.VMEM_SHARED((N_NODES,), jnp.float32),
    ],
)
def _sc_deg(dst_hbm, zeros_hbm, ones_hbm, out_hbm, idx_v, ones_v, deg_sh):
    c = lax.axis_index("c")
    s = lax.axis_index("s")
    wid = s * NC + c
    @pl.when(s == 0)
    def _():
        pltpu.sync_copy(zeros_hbm, deg_sh)
    pltpu.sync_copy(ones_hbm, ones_v)
    plsc.subcore_barrier()

    def chunk(ci, carry):
        base = wid * EPT + ci * C
        pltpu.sync_copy(dst_hbm.at[pl.ds(base, C)], idx_v)
        pltpu.sync_copy(ones_v, deg_sh.at[idx_v], add=True)
        return carry

    lax.fori_loop(0, NCHUNK, chunk, 0)
    plsc.subcore_barrier()
    @pl.when(s == 0)
    def _():
        pltpu.sync_copy(deg_sh, out_hbm.at[pl.ds(c * N_NODES, N_NODES)])


@functools.partial(
    pl.kernel,
    out_type=jax.ShapeDtypeStruct((NC * N_NODES, D), jnp.float32),
    mesh=_mesh,
    scratch_types=[
        pltpu.VMEM((C,), jnp.int32),
        pltpu.VMEM((C,), jnp.int32),
        pltpu.VMEM((C, D), jnp.float32),
        pltpu.VMEM_SHARED((N_NODES, D), jnp.float32),
        pltpu.SemaphoreType.DMA,
    ],
)
def _sc_edge(h_hbm, src_hbm, dst_hbm, zrows_hbm, out_hbm,
             src_v, dst_v, rows_v, acc_sh, sem):
    c = lax.axis_index("c")
    s = lax.axis_index("s")
    wid = s * NC + c
    pltpu.sync_copy(zrows_hbm.at[pl.ds(s * ROWS_PT, ROWS_PT)],
                    acc_sh.at[pl.ds(s * ROWS_PT, ROWS_PT)])
    plsc.subcore_barrier()

    def chunk(ci, carry):
        base = wid * EPT + ci * C
        pltpu.sync_copy(src_hbm.at[pl.ds(base, C)], src_v)
        pltpu.sync_copy(dst_hbm.at[pl.ds(base, C)], dst_v)
        pltpu.async_copy(h_hbm.at[src_v], rows_v, sem).wait()
        pltpu.sync_copy(rows_v, acc_sh.at[dst_v], add=True)
        return carry

    lax.fori_loop(0, NCHUNK, chunk, 0)
    plsc.subcore_barrier()
    pltpu.sync_copy(acc_sh.at[pl.ds(s * ROWS_PT, ROWS_PT)],
                    out_hbm.at[pl.ds(c * N_NODES + s * ROWS_PT, ROWS_PT)])


B = 1000


def _dinv_of(degT_blk):
    deg = degT_blk[:, 0] + degT_blk[:, 1] + 1.0
    return lax.rsqrt(deg)


def _tc_h1_body(degT_ref, x_ref, w_ref, h1p_ref):
    dinv = _dinv_of(degT_ref[...])
    h = jnp.dot(x_ref[...], w_ref[...], preferred_element_type=jnp.float32)
    h1p_ref[...] = h * dinv[:, None]


def _tc_mid_body(degT_ref, s_ref, h1p_ref, b1_ref, w2_ref, h1o_ref, h2p_ref):
    dinv = _dinv_of(degT_ref[...])
    agg = s_ref[0] + s_ref[1] + h1p_ref[...]
    h1o = jnp.maximum(dinv[:, None] * agg + b1_ref[...], 0.0)
    h1o_ref[...] = h1o
    h2 = jnp.dot(h1o, w2_ref[...], preferred_element_type=jnp.float32)
    h2p_ref[...] = h2 * dinv[:, None]


def _tc_fin_body(degT_ref, s_ref, h2p_ref, b2_ref, h1o_ref, out_ref):
    dinv = _dinv_of(degT_ref[...])
    agg = s_ref[0] + s_ref[1] + h2p_ref[...]
    h2o = jnp.maximum(dinv[:, None] * agg + b2_ref[...], 0.0)
    out_ref[:, :D] = h1o_ref[...]
    out_ref[:, D:] = h2o


_degT_spec = pl.BlockSpec((B, NC), lambda i: (i, 0))
_row_spec = pl.BlockSpec((B, D), lambda i: (i, 0))
_S_spec = pl.BlockSpec((NC, B, D), lambda i: (0, i, 0))
_w_spec = pl.BlockSpec((D, D), lambda i: (0, 0))
_b_spec = pl.BlockSpec((1, D), lambda i: (0, 0))

_GRID = (N_NODES // B,)

_tc_h1 = pl.pallas_call(
    _tc_h1_body,
    grid=_GRID,
    in_specs=[_degT_spec, _row_spec, _w_spec],
    out_specs=_row_spec,
    out_shape=jax.ShapeDtypeStruct((N_NODES, D), jnp.float32),
)

_tc_mid = pl.pallas_call(
    _tc_mid_body,
    grid=_GRID,
    in_specs=[_degT_spec, _S_spec, _row_spec, _b_spec, _w_spec],
    out_specs=[_row_spec, _row_spec],
    out_shape=[jax.ShapeDtypeStruct((N_NODES, D), jnp.float32),
               jax.ShapeDtypeStruct((N_NODES, D), jnp.float32)],
)

_tc_fin = pl.pallas_call(
    _tc_fin_body,
    grid=_GRID,
    in_specs=[_degT_spec, _S_spec, _row_spec, _b_spec, _row_spec],
    out_specs=pl.BlockSpec((B, 2 * D), lambda i: (i, 0)),
    out_shape=jax.ShapeDtypeStruct((N_NODES, 2 * D), jnp.float32),
)


def kernel(x, edge_index, W1, b1, W2, b2):
    ei = edge_index.astype(jnp.int32)
    src = ei[0]
    dst = ei[1]
    zeros_n = jnp.zeros((N_NODES,), jnp.float32)
    ones_c = jnp.ones((C,), jnp.float32)
    zrows = jnp.zeros((N_NODES, D), jnp.float32)
    b1r = b1.reshape(1, D)
    b2r = b2.reshape(1, D)

    deg_parts = _sc_deg(dst, zeros_n, ones_c)
    degT = deg_parts.reshape(NC, N_NODES).T

    h1p = _tc_h1(degT, x, W1)
    S1 = _sc_edge(h1p, src, dst, zrows).reshape(NC, N_NODES, D)
    h1o, h2p = _tc_mid(degT, S1, h1p, b1r, W2)
    S2 = _sc_edge(h2p, src, dst, zrows).reshape(NC, N_NODES, D)
    out = _tc_fin(degT, S2, h2p, b2r, h1o)
    return out


# speedup vs baseline: 12.8897x; 12.8897x over previous
import functools

import jax
import jax.numpy as jnp
from jax import lax
from jax.experimental import pallas as pl
from jax.experimental.pallas import tpu as pltpu
from jax.experimental.pallas import tpu_sc as plsc

N_NODES = 10000
N_EDGES = 320000
D = 128

NC = 2
NS = 16
NW = NC * NS
EPT = N_EDGES // NW
C = 80
NCHUNK = EPT // C
RPT = 640
NPAD = NS * RPT
RCH = 80
FULL_CH = RPT // RCH
LAST_CH = (N_NODES - (NS - 1) * RPT) // RCH

_mesh = plsc.VectorSubcoreMesh(core_axis_name="c", subcore_axis_name="s")


@functools.partial(
    pl.kernel,
    out_type=jax.ShapeDtypeStruct((NC * N_NODES,), jnp.float32),
    mesh=_mesh,
    scratch_types=[
        pltpu.VMEM((C,), jnp.int32),
        pltpu.VMEM((C,), jnp.float32),
        pltpu.VMEM((RCH,), jnp.float32),
        pltpu.VMEM_SHARED((NPAD,), jnp.float32),
    ],
)
def _sc_deg(dst_hbm, z80_hbm, ones_hbm, out_hbm, idx_v, ones_v, db_v, deg_sh):
    c = lax.axis_index("c")
    s = lax.axis_index("s")
    wid = s * NC + c
    pltpu.sync_copy(z80_hbm, db_v)

    def initk(k, carry):
        @pl.when((s < NS - 1) | (k < LAST_CH))
        def _():
            pltpu.sync_copy(db_v, deg_sh.at[pl.ds(s * RPT + k * RCH, RCH)])
        return carry

    lax.fori_loop(0, FULL_CH, initk, 0)
    pltpu.sync_copy(ones_hbm, ones_v)
    plsc.subcore_barrier()

    def chunk(ci, carry):
        base = wid * EPT + ci * C
        pltpu.sync_copy(dst_hbm.at[pl.ds(base, C)], idx_v)
        pltpu.sync_copy(ones_v, deg_sh.at[idx_v], add=True)
        return carry

    lax.fori_loop(0, NCHUNK, chunk, 0)
    plsc.subcore_barrier()

    def draink(k, carry):
        @pl.when((s < NS - 1) | (k < LAST_CH))
        def _():
            pltpu.sync_copy(deg_sh.at[pl.ds(s * RPT + k * RCH, RCH)], db_v)
            pltpu.sync_copy(
                db_v, out_hbm.at[pl.ds(c * N_NODES + s * RPT + k * RCH, RCH)])
        return carry

    lax.fori_loop(0, FULL_CH, draink, 0)


@functools.partial(
    pl.kernel,
    out_type=jax.ShapeDtypeStruct((NC * N_NODES, D), jnp.float32),
    mesh=_mesh,
    scratch_types=[
        pltpu.VMEM((C,), jnp.int32),
        pltpu.VMEM((C,), jnp.int32),
        pltpu.VMEM((C, D), jnp.float32),
        pltpu.VMEM((RCH, D), jnp.float32),
        pltpu.VMEM_SHARED((NPAD, D), jnp.float32),
        pltpu.SemaphoreType.DMA,
    ],
)
def _sc_edge(h_hbm, src_hbm, dst_hbm, zrows_hbm, out_hbm,
             src_v, dst_v, rows_v, zb_v, acc_sh, sem):
    c = lax.axis_index("c")
    s = lax.axis_index("s")
    wid = s * NC + c
    pltpu.sync_copy(zrows_hbm, zb_v)

    def initk(k, carry):
        @pl.when((s < NS - 1) | (k < LAST_CH))
        def _():
            pltpu.sync_copy(zb_v, acc_sh.at[pl.ds(s * RPT + k * RCH, RCH)])
        return carry

    lax.fori_loop(0, FULL_CH, initk, 0)
    plsc.subcore_barrier()

    def chunk(ci, carry):
        base = wid * EPT + ci * C
        pltpu.sync_copy(src_hbm.at[pl.ds(base, C)], src_v)
        pltpu.sync_copy(dst_hbm.at[pl.ds(base, C)], dst_v)
        pltpu.async_copy(h_hbm.at[src_v], rows_v, sem).wait()
        pltpu.sync_copy(rows_v, acc_sh.at[dst_v], add=True)
        return carry

    lax.fori_loop(0, NCHUNK, chunk, 0)
    plsc.subcore_barrier()

    def draink(k, carry):
        @pl.when((s < NS - 1) | (k < LAST_CH))
        def _():
            pltpu.sync_copy(acc_sh.at[pl.ds(s * RPT + k * RCH, RCH)], zb_v)
            pltpu.sync_copy(
                zb_v,
                out_hbm.at[pl.ds(c * N_NODES + s * RPT + k * RCH, RCH)])
        return carry

    lax.fori_loop(0, FULL_CH, draink, 0)


B = 1000


def _dinv_of(degT_blk):
    deg = degT_blk[:, 0] + degT_blk[:, 1] + 1.0
    return lax.rsqrt(deg)


def _tc_h1_body(degT_ref, x_ref, w_ref, h1p_ref):
    dinv = _dinv_of(degT_ref[...])
    h = jnp.dot(x_ref[...], w_ref[...], preferred_element_type=jnp.float32)
    h1p_ref[...] = h * dinv[:, None]


def _tc_mid_body(degT_ref, s_ref, h1p_ref, b1_ref, w2_ref, h1o_ref, h2p_ref):
    dinv = _dinv_of(degT_ref[...])
    agg = s_ref[0] + s_ref[1] + h1p_ref[...]
    h1o = jnp.maximum(dinv[:, None] * agg + b1_ref[...], 0.0)
    h1o_ref[...] = h1o
    h2 = jnp.dot(h1o, w2_ref[...], preferred_element_type=jnp.float32)
    h2p_ref[...] = h2 * dinv[:, None]


def _tc_fin_body(degT_ref, s_ref, h2p_ref, b2_ref, h1o_ref, out_ref):
    dinv = _dinv_of(degT_ref[...])
    agg = s_ref[0] + s_ref[1] + h2p_ref[...]
    h2o = jnp.maximum(dinv[:, None] * agg + b2_ref[...], 0.0)
    out_ref[:, :D] = h1o_ref[...]
    out_ref[:, D:] = h2o


_degT_spec = pl.BlockSpec((B, NC), lambda i: (i, 0))
_row_spec = pl.BlockSpec((B, D), lambda i: (i, 0))
_S_spec = pl.BlockSpec((NC, B, D), lambda i: (0, i, 0))
_w_spec = pl.BlockSpec((D, D), lambda i: (0, 0))
_b_spec = pl.BlockSpec((1, D), lambda i: (0, 0))

_GRID = (N_NODES // B,)

_tc_h1 = pl.pallas_call(
    _tc_h1_body,
    grid=_GRID,
    in_specs=[_degT_spec, _row_spec, _w_spec],
    out_specs=_row_spec,
    out_shape=jax.ShapeDtypeStruct((N_NODES, D), jnp.float32),
)

_tc_mid = pl.pallas_call(
    _tc_mid_body,
    grid=_GRID,
    in_specs=[_degT_spec, _S_spec, _row_spec, _b_spec, _w_spec],
    out_specs=[_row_spec, _row_spec],
    out_shape=[jax.ShapeDtypeStruct((N_NODES, D), jnp.float32),
               jax.ShapeDtypeStruct((N_NODES, D), jnp.float32)],
)

_tc_fin = pl.pallas_call(
    _tc_fin_body,
    grid=_GRID,
    in_specs=[_degT_spec, _S_spec, _row_spec, _b_spec, _row_spec],
    out_specs=pl.BlockSpec((B, 2 * D), lambda i: (i, 0)),
    out_shape=jax.ShapeDtypeStruct((N_NODES, 2 * D), jnp.float32),
)


def kernel(x, edge_index, W1, b1, W2, b2):
    ei = edge_index.astype(jnp.int32)
    src = ei[0]
    dst = ei[1]
    zeros_n = jnp.zeros((RCH,), jnp.float32)
    ones_c = jnp.ones((C,), jnp.float32)
    zrows = jnp.zeros((RCH, D), jnp.float32)
    b1r = b1.reshape(1, D)
    b2r = b2.reshape(1, D)

    deg_parts = _sc_deg(dst, zeros_n, ones_c)
    degT = deg_parts.reshape(NC, N_NODES).T

    h1p = _tc_h1(degT, x, W1)
    S1 = _sc_edge(h1p, src, dst, zrows).reshape(NC, N_NODES, D)
    h1o, h2p = _tc_mid(degT, S1, h1p, b1r, W2)
    S2 = _sc_edge(h2p, src, dst, zrows).reshape(NC, N_NODES, D)
    out = _tc_fin(degT, S2, h2p, b2r, h1o)
    return out


# speedup vs baseline: 15.3533x; 1.1911x over previous
import functools

import jax
import jax.numpy as jnp
from jax import lax
from jax.experimental import pallas as pl
from jax.experimental.pallas import tpu as pltpu
from jax.experimental.pallas import tpu_sc as plsc

N_NODES = 10000
N_EDGES = 320000
D = 128

NC = 2
NS = 16
NW = NC * NS
EPT = N_EDGES // NW
CP = 128
NCHP = -(-EPT // CP)
EPTP = CP * NCHP
PADW = EPTP - EPT
RPT = 640
NPAD = NS * RPT
RCH = 80
FULL_CH = RPT // RCH
LAST_CH = (N_NODES - (NS - 1) * RPT) // RCH

NPADE = NCHP * CP
NB = 2
NGRP = NCHP // NB
NTAIL = NCHP - NGRP * NB
NBD = 4
NGRPD = NCHP // NBD
NTAILD = NCHP - NGRPD * NBD
NFR = N_NODES // CP
LASTR = N_NODES - NFR * CP
NRSLOT = -(-(NFR + 1) // NS)

_mesh = plsc.VectorSubcoreMesh(core_axis_name="c", subcore_axis_name="s")


@functools.partial(
    pl.kernel,
    out_type=jax.ShapeDtypeStruct((NC * N_NODES,), jnp.float32),
    mesh=_mesh,
    scratch_types=(
        [pltpu.VMEM((CP,), jnp.float32),
         pltpu.VMEM((RCH,), jnp.float32)]
        + [pltpu.VMEM((CP,), jnp.int32)] * NBD
        + [pltpu.VMEM_SHARED((NPAD,), jnp.float32)]
        + [pltpu.SemaphoreType.DMA] * (2 * NBD)
    ),
)
def _sc_deg(dst_hbm, z80_hbm, ones_hbm, out_hbm, ones_v, db_v, *rest):
    didx = rest[:NBD]
    deg_sh = rest[NBD]
    xs = rest[NBD + 1:NBD + 1 + NBD]
    ss = rest[NBD + 1 + NBD:]
    c = lax.axis_index("c")
    s = lax.axis_index("s")
    wid = s * NC + c
    pltpu.sync_copy(z80_hbm, db_v)

    def initk(k, carry):
        @pl.when((s < NS - 1) | (k < LAST_CH))
        def _():
            pltpu.sync_copy(db_v, deg_sh.at[pl.ds(s * RPT + k * RCH, RCH)])
        return carry

    lax.fori_loop(0, FULL_CH, initk, 0)
    pltpu.sync_copy(ones_hbm, ones_v)
    for b in range(NBD):
        pltpu.async_copy(dst_hbm.at[pl.ds(wid * EPTP + b * CP, CP)], didx[b],
                         xs[b])
    plsc.subcore_barrier()

    def group(g, carry):
        for b in range(NBD):
            ci = g * NBD + b

            @pl.when(g > 0)
            def _():
                pltpu.make_async_copy(ones_hbm, ones_v, ss[b]).wait()
                pltpu.async_copy(dst_hbm.at[pl.ds(wid * EPTP + ci * CP, CP)],
                                 didx[b], xs[b])
        for b in range(NBD):
            pltpu.make_async_copy(dst_hbm.at[pl.ds(0, CP)], didx[b],
                                  xs[b]).wait()
            pltpu.async_copy(ones_v, deg_sh.at[didx[b]], ss[b], add=True)
        return carry

    lax.fori_loop(0, NGRPD, group, 0)
    for b in range(NBD):
        pltpu.make_async_copy(ones_hbm, ones_v, ss[b]).wait()
    for t in range(NTAILD):
        ci = NGRPD * NBD + t
        pltpu.sync_copy(dst_hbm.at[pl.ds(wid * EPTP + ci * CP, CP)], didx[t])
        pltpu.sync_copy(ones_v, deg_sh.at[didx[t]], add=True)
    plsc.subcore_barrier()

    def draink(k, carry):
        @pl.when((s < NS - 1) | (k < LAST_CH))
        def _():
            pltpu.sync_copy(deg_sh.at[pl.ds(s * RPT + k * RCH, RCH)], db_v)
            pltpu.sync_copy(
                db_v, out_hbm.at[pl.ds(c * N_NODES + s * RPT + k * RCH, RCH)])
        return carry

    lax.fori_loop(0, FULL_CH, draink, 0)


_edge_scratch = (
    [pltpu.VMEM_SHARED((NPADE, D), jnp.float32),
     pltpu.VMEM((LASTR, D), jnp.float32)]
    + [pltpu.VMEM((CP, D), jnp.float32)] * NB
    + [pltpu.VMEM((CP,), jnp.int32)] * NB
    + [pltpu.VMEM((CP,), jnp.int32)] * NB
    + [pltpu.SemaphoreType.DMA] * (4 * NB)
)


@functools.partial(
    pl.kernel,
    out_type=jax.ShapeDtypeStruct((NC * N_NODES, D), jnp.float32),
    mesh=_mesh,
    scratch_types=_edge_scratch,
)
def _sc_edge(h_hbm, src_hbm, dst_hbm, zrows_hbm, out_hbm,
             acc_sh, zb16, *ring):
    rows = ring[:NB]
    sidx = ring[NB:2 * NB]
    didx = ring[2 * NB:3 * NB]
    gs = ring[3 * NB:4 * NB]
    ss = ring[4 * NB:5 * NB]
    xs = ring[5 * NB:6 * NB]
    ds_ = ring[6 * NB:]
    zb_v = rows[0]
    c = lax.axis_index("c")
    s = lax.axis_index("s")
    wid = s * NC + c
    pltpu.sync_copy(zrows_hbm, zb_v)
    pltpu.sync_copy(zrows_hbm.at[pl.ds(0, LASTR)], zb16)

    def initk(k, carry):
        j = k * NS + s

        @pl.when(j < NFR)
        def _():
            pltpu.sync_copy(zb_v, acc_sh.at[pl.ds(j * CP, CP)])

        @pl.when(j == NFR)
        def _():
            pltpu.sync_copy(zb16, acc_sh.at[pl.ds(NFR * CP, LASTR)])
        return carry

    lax.fori_loop(0, NRSLOT, initk, 0)
    for b in range(NB):
        pltpu.async_copy(src_hbm.at[pl.ds(wid * EPTP + b * CP, CP)], sidx[b],
                         xs[b])
        pltpu.async_copy(dst_hbm.at[pl.ds(wid * EPTP + b * CP, CP)], didx[b],
                         ds_[b])
    plsc.subcore_barrier()

    def group(g, carry):
        for b in range(NB):
            ci = g * NB + b

            @pl.when(g > 0)
            def _():
                pltpu.make_async_copy(
                    h_hbm.at[pl.ds(0, CP)], rows[b], ss[b]).wait()
                pltpu.async_copy(dst_hbm.at[pl.ds(wid * EPTP + ci * CP, CP)],
                                 didx[b], ds_[b])
            pltpu.make_async_copy(src_hbm.at[pl.ds(0, CP)], sidx[b],
                                  xs[b]).wait()
            pltpu.async_copy(h_hbm.at[sidx[b]], rows[b], gs[b])
        for b in range(NB):
            ci = g * NB + b
            pltpu.make_async_copy(h_hbm.at[pl.ds(0, CP)], rows[b],
                                  gs[b]).wait()

            @pl.when(ci + NB < NCHP)
            def _():
                pltpu.async_copy(
                    src_hbm.at[pl.ds(wid * EPTP + (ci + NB) * CP, CP)],
                    sidx[b], xs[b])
            pltpu.make_async_copy(dst_hbm.at[pl.ds(0, CP)], didx[b],
                                  ds_[b]).wait()
            pltpu.async_copy(rows[b], acc_sh.at[didx[b]], ss[b], add=True)
        return carry

    lax.fori_loop(0, NGRP, group, 0)
    for b in range(NB):
        pltpu.make_async_copy(h_hbm.at[pl.ds(0, CP)], rows[b], ss[b]).wait()
    for t in range(NTAIL):
        ci = NGRP * NB + t
        pltpu.make_async_copy(src_hbm.at[pl.ds(0, CP)], sidx[t], xs[t]).wait()
        pltpu.sync_copy(dst_hbm.at[pl.ds(wid * EPTP + ci * CP, CP)], didx[t])
        pltpu.async_copy(h_hbm.at[sidx[t]], rows[t], gs[t])
        pltpu.make_async_copy(h_hbm.at[pl.ds(0, CP)], rows[t], gs[t]).wait()
        pltpu.sync_copy(rows[t], acc_sh.at[didx[t]], add=True)
    plsc.subcore_barrier()

    def draink(k, carry):
        j = k * NS + s

        @pl.when(j < NFR)
        def _():
            pltpu.sync_copy(acc_sh.at[pl.ds(j * CP, CP)], zb_v)
            pltpu.sync_copy(zb_v,
                            out_hbm.at[pl.ds(c * N_NODES + j * CP, CP)])

        @pl.when(j == NFR)
        def _():
            pltpu.sync_copy(acc_sh.at[pl.ds(NFR * CP, LASTR)], zb16)
            pltpu.sync_copy(
                zb16, out_hbm.at[pl.ds(c * N_NODES + NFR * CP, LASTR)])
        return carry

    lax.fori_loop(0, NRSLOT, draink, 0)


B = 1000


def _dinv_of(degT_blk):
    deg = degT_blk[:, 0] + degT_blk[:, 1] + 1.0
    return lax.rsqrt(deg)


def _tc_h1_body(degT_ref, x_ref, w_ref, h1p_ref):
    dinv = _dinv_of(degT_ref[...])
    h = jnp.dot(x_ref[...], w_ref[...], preferred_element_type=jnp.float32)
    h1p_ref[...] = h * dinv[:, None]


def _tc_mid_body(degT_ref, s_ref, h1p_ref, b1_ref, w2_ref, h1o_ref, h2p_ref):
    dinv = _dinv_of(degT_ref[...])
    agg = s_ref[0] + s_ref[1] + h1p_ref[...]
    h1o = jnp.maximum(dinv[:, None] * agg + b1_ref[...], 0.0)
    h1o_ref[...] = h1o
    h2 = jnp.dot(h1o, w2_ref[...], preferred_element_type=jnp.float32)
    h2p_ref[...] = h2 * dinv[:, None]


def _tc_fin_body(degT_ref, s_ref, h2p_ref, b2_ref, h1o_ref, out_ref):
    dinv = _dinv_of(degT_ref[...])
    agg = s_ref[0] + s_ref[1] + h2p_ref[...]
    h2o = jnp.maximum(dinv[:, None] * agg + b2_ref[...], 0.0)
    out_ref[:, :D] = h1o_ref[...]
    out_ref[:, D:] = h2o


_degT_spec = pl.BlockSpec((B, NC), lambda i: (i, 0))
_row_spec = pl.BlockSpec((B, D), lambda i: (i, 0))
_S_spec = pl.BlockSpec((NC, B, D), lambda i: (0, i, 0))
_w_spec = pl.BlockSpec((D, D), lambda i: (0, 0))
_b_spec = pl.BlockSpec((1, D), lambda i: (0, 0))

_GRID = (N_NODES // B,)

_tc_h1 = pl.pallas_call(
    _tc_h1_body,
    grid=_GRID,
    in_specs=[_degT_spec, _row_spec, _w_spec],
    out_specs=_row_spec,
    out_shape=jax.ShapeDtypeStruct((N_NODES, D), jnp.float32),
)

_tc_mid = pl.pallas_call(
    _tc_mid_body,
    grid=_GRID,
    in_specs=[_degT_spec, _S_spec, _row_spec, _b_spec, _w_spec],
    out_specs=[_row_spec, _row_spec],
    out_shape=[jax.ShapeDtypeStruct((N_NODES, D), jnp.float32),
               jax.ShapeDtypeStruct((N_NODES, D), jnp.float32)],
)

_tc_fin = pl.pallas_call(
    _tc_fin_body,
    grid=_GRID,
    in_specs=[_degT_spec, _S_spec, _row_spec, _b_spec, _row_spec],
    out_specs=pl.BlockSpec((B, 2 * D), lambda i: (i, 0)),
    out_shape=jax.ShapeDtypeStruct((N_NODES, 2 * D), jnp.float32),
)


def kernel(x, edge_index, W1, b1, W2, b2):
    ei = edge_index.astype(jnp.int32)
    srcp = jnp.concatenate(
        [ei[0].reshape(NW, EPT), jnp.zeros((NW, PADW), jnp.int32)],
        axis=1).reshape(-1)
    padtgt = N_NODES + (jnp.arange(PADW, dtype=jnp.int32)
                        % (NPADE - N_NODES))
    dstp = jnp.concatenate(
        [ei[1].reshape(NW, EPT), jnp.broadcast_to(padtgt, (NW, PADW))],
        axis=1).reshape(-1)
    zeros_n = jnp.zeros((RCH,), jnp.float32)
    ones_c = jnp.ones((CP,), jnp.float32)
    zrows = jnp.zeros((CP, D), jnp.float32)
    b1r = b1.reshape(1, D)
    b2r = b2.reshape(1, D)

    deg_parts = _sc_deg(dstp, zeros_n, ones_c)
    degT = deg_parts.reshape(NC, N_NODES).T

    h1p = _tc_h1(degT, x, W1)
    S1 = _sc_edge(h1p, srcp, dstp, zrows).reshape(NC, N_NODES, D)
    h1o, h2p = _tc_mid(degT, S1, h1p, b1r, W2)
    S2 = _sc_edge(h2p, srcp, dstp, zrows).reshape(NC, N_NODES, D)
    out = _tc_fin(degT, S2, h2p, b2r, h1o)
    return out


# speedup vs baseline: 16.7011x; 1.0878x over previous
import functools

import jax
import jax.numpy as jnp
from jax import lax
from jax.experimental import pallas as pl
from jax.experimental.pallas import tpu as pltpu
from jax.experimental.pallas import tpu_sc as plsc

N_NODES = 10000
N_EDGES = 320000
D = 128

NC = 2
NS = 16
NW = NC * NS
EPT = N_EDGES // NW
CP = 128
NCHP = -(-EPT // CP)
EPTP = CP * NCHP
PADW = EPTP - EPT
NPADE = N_NODES + 16
RPT = 640
NPAD = NS * RPT
RCH = 80
FULL_CH = RPT // RCH
LAST_CH = (N_NODES - (NS - 1) * RPT) // RCH

NB = 3
NGRP = NCHP // NB
NTAIL = NCHP - NGRP * NB
NBD = 4
NGRPD = NCHP // NBD
NTAILD = NCHP - NGRPD * NBD
NFR = N_NODES // CP
LASTR = N_NODES - NFR * CP
NRSLOT = -(-(NFR + 1) // NS)

_mesh = plsc.VectorSubcoreMesh(core_axis_name="c", subcore_axis_name="s")


@functools.partial(
    pl.kernel,
    out_type=jax.ShapeDtypeStruct((NC * N_NODES,), jnp.float32),
    mesh=_mesh,
    scratch_types=(
        [pltpu.VMEM((CP,), jnp.float32),
         pltpu.VMEM((RCH,), jnp.float32)]
        + [pltpu.VMEM((CP,), jnp.int32)] * NBD
        + [pltpu.VMEM_SHARED((NPAD,), jnp.float32)]
        + [pltpu.SemaphoreType.DMA] * (2 * NBD)
    ),
)
def _sc_deg(dst_hbm, z80_hbm, ones_hbm, out_hbm, ones_v, db_v, *rest):
    didx = rest[:NBD]
    deg_sh = rest[NBD]
    xs = rest[NBD + 1:NBD + 1 + NBD]
    ss = rest[NBD + 1 + NBD:]
    c = lax.axis_index("c")
    s = lax.axis_index("s")
    wid = s * NC + c
    pltpu.sync_copy(z80_hbm, db_v)

    def initk(k, carry):
        @pl.when((s < NS - 1) | (k < LAST_CH))
        def _():
            pltpu.sync_copy(db_v, deg_sh.at[pl.ds(s * RPT + k * RCH, RCH)])
        return carry

    lax.fori_loop(0, FULL_CH, initk, 0)
    pltpu.sync_copy(ones_hbm, ones_v)
    for b in range(NBD):
        pltpu.async_copy(dst_hbm.at[pl.ds(wid * EPTP + b * CP, CP)], didx[b],
                         xs[b])
    plsc.subcore_barrier()

    def group(g, carry):
        for b in range(NBD):
            ci = g * NBD + b

            @pl.when(g > 0)
            def _():
                pltpu.make_async_copy(ones_hbm, ones_v, ss[b]).wait()
                pltpu.async_copy(dst_hbm.at[pl.ds(wid * EPTP + ci * CP, CP)],
                                 didx[b], xs[b])
        for b in range(NBD):
            pltpu.make_async_copy(dst_hbm.at[pl.ds(0, CP)], didx[b],
                                  xs[b]).wait()
            pltpu.async_copy(ones_v, deg_sh.at[didx[b]], ss[b], add=True)
        return carry

    lax.fori_loop(0, NGRPD, group, 0)
    for b in range(NBD):
        pltpu.make_async_copy(ones_hbm, ones_v, ss[b]).wait()
    for t in range(NTAILD):
        ci = NGRPD * NBD + t
        pltpu.sync_copy(dst_hbm.at[pl.ds(wid * EPTP + ci * CP, CP)], didx[t])
        pltpu.sync_copy(ones_v, deg_sh.at[didx[t]], add=True)
    plsc.subcore_barrier()

    def draink(k, carry):
        @pl.when((s < NS - 1) | (k < LAST_CH))
        def _():
            pltpu.sync_copy(deg_sh.at[pl.ds(s * RPT + k * RCH, RCH)], db_v)
            pltpu.sync_copy(
                db_v, out_hbm.at[pl.ds(c * N_NODES + s * RPT + k * RCH, RCH)])
        return carry

    lax.fori_loop(0, FULL_CH, draink, 0)


_edge_scratch = (
    [pltpu.VMEM_SHARED((NPADE, D), jnp.float32)]
    + [pltpu.VMEM((CP, D), jnp.float32)] * NB
    + [pltpu.VMEM((CP,), jnp.int32)] * NB
    + [pltpu.VMEM((CP,), jnp.int32)] * NB
    + [pltpu.SemaphoreType.DMA] * (4 * NB)
)


@functools.partial(
    pl.kernel,
    out_type=jax.ShapeDtypeStruct((NC * N_NODES, D), jnp.float32),
    mesh=_mesh,
    scratch_types=_edge_scratch,
)
def _sc_edge(h_hbm, src_hbm, dst_hbm, zrows_hbm, out_hbm, acc_sh, *ring):
    rows = ring[:NB]
    sidx = ring[NB:2 * NB]
    didx = ring[2 * NB:3 * NB]
    gs = ring[3 * NB:4 * NB]
    ss = ring[4 * NB:5 * NB]
    xs = ring[5 * NB:6 * NB]
    ds_ = ring[6 * NB:]
    zb_v = rows[0]
    c = lax.axis_index("c")
    s = lax.axis_index("s")
    wid = s * NC + c
    pltpu.sync_copy(zrows_hbm, zb_v)

    def initk(k, carry):
        j = k * NS + s

        @pl.when(j < NFR)
        def _():
            pltpu.sync_copy(zb_v, acc_sh.at[pl.ds(j * CP, CP)])

        @pl.when(j == NFR)
        def _():
            pltpu.sync_copy(zb_v.at[pl.ds(0, LASTR)],
                            acc_sh.at[pl.ds(NFR * CP, LASTR)])
        return carry

    lax.fori_loop(0, NRSLOT, initk, 0)
    for b in range(NB):
        pltpu.async_copy(src_hbm.at[pl.ds(wid * EPTP + b * CP, CP)], sidx[b],
                         xs[b])
        pltpu.async_copy(dst_hbm.at[pl.ds(wid * EPTP + b * CP, CP)], didx[b],
                         ds_[b])
    plsc.subcore_barrier()

    def group(g, carry):
        for b in range(NB):
            ci = g * NB + b

            @pl.when(g > 0)
            def _():
                pltpu.make_async_copy(
                    h_hbm.at[pl.ds(0, CP)], rows[b], ss[b]).wait()
                pltpu.async_copy(dst_hbm.at[pl.ds(wid * EPTP + ci * CP, CP)],
                                 didx[b], ds_[b])
            pltpu.make_async_copy(src_hbm.at[pl.ds(0, CP)], sidx[b],
                                  xs[b]).wait()
            pltpu.async_copy(h_hbm.at[sidx[b]], rows[b], gs[b])
        for b in range(NB):
            ci = g * NB + b
            pltpu.make_async_copy(h_hbm.at[pl.ds(0, CP)], rows[b],
                                  gs[b]).wait()

            @pl.when(ci + NB < NCHP)
            def _():
                pltpu.async_copy(
                    src_hbm.at[pl.ds(wid * EPTP + (ci + NB) * CP, CP)],
                    sidx[b], xs[b])
            pltpu.make_async_copy(dst_hbm.at[pl.ds(0, CP)], didx[b],
                                  ds_[b]).wait()
            pltpu.async_copy(rows[b], acc_sh.at[didx[b]], ss[b], add=True)
        return carry

    lax.fori_loop(0, NGRP, group, 0)
    for b in range(NB):
        pltpu.make_async_copy(h_hbm.at[pl.ds(0, CP)], rows[b], ss[b]).wait()
    for t in range(NTAIL):
        ci = NGRP * NB + t
        pltpu.make_async_copy(src_hbm.at[pl.ds(0, CP)], sidx[t], xs[t]).wait()
        pltpu.sync_copy(dst_hbm.at[pl.ds(wid * EPTP + ci * CP, CP)], didx[t])
        pltpu.async_copy(h_hbm.at[sidx[t]], rows[t], gs[t])
        pltpu.make_async_copy(h_hbm.at[pl.ds(0, CP)], rows[t], gs[t]).wait()
        pltpu.sync_copy(rows[t], acc_sh.at[didx[t]], add=True)
    plsc.subcore_barrier()

    def draink(k, carry):
        j = k * NS + s

        @pl.when(j < NFR)
        def _():
            pltpu.sync_copy(acc_sh.at[pl.ds(j * CP, CP)], zb_v)
            pltpu.sync_copy(zb_v,
                            out_hbm.at[pl.ds(c * N_NODES + j * CP, CP)])

        @pl.when(j == NFR)
        def _():
            pltpu.sync_copy(acc_sh.at[pl.ds(NFR * CP, LASTR)],
                            rows[1].at[pl.ds(0, LASTR)])
            pltpu.sync_copy(
                rows[1].at[pl.ds(0, LASTR)],
                out_hbm.at[pl.ds(c * N_NODES + NFR * CP, LASTR)])
        return carry

    lax.fori_loop(0, NRSLOT, draink, 0)


B = 1000


def _dinv_of(degT_blk):
    deg = degT_blk[:, 0] + degT_blk[:, 1] + 1.0
    return lax.rsqrt(deg)


def _tc_h1_body(degT_ref, x_ref, w_ref, h1p_ref):
    dinv = _dinv_of(degT_ref[...])
    h = jnp.dot(x_ref[...], w_ref[...], preferred_element_type=jnp.float32)
    h1p_ref[...] = h * dinv[:, None]


def _tc_mid_body(degT_ref, s_ref, h1p_ref, b1_ref, w2_ref, h1o_ref, h2p_ref):
    dinv = _dinv_of(degT_ref[...])
    agg = s_ref[0] + s_ref[1] + h1p_ref[...]
    h1o = jnp.maximum(dinv[:, None] * agg + b1_ref[...], 0.0)
    h1o_ref[...] = h1o
    h2 = jnp.dot(h1o, w2_ref[...], preferred_element_type=jnp.float32)
    h2p_ref[...] = h2 * dinv[:, None]


def _tc_fin_body(degT_ref, s_ref, h2p_ref, b2_ref, h1o_ref, out_ref):
    dinv = _dinv_of(degT_ref[...])
    agg = s_ref[0] + s_ref[1] + h2p_ref[...]
    h2o = jnp.maximum(dinv[:, None] * agg + b2_ref[...], 0.0)
    out_ref[:, :D] = h1o_ref[...]
    out_ref[:, D:] = h2o


_degT_spec = pl.BlockSpec((B, NC), lambda i: (i, 0))
_row_spec = pl.BlockSpec((B, D), lambda i: (i, 0))
_S_spec = pl.BlockSpec((NC, B, D), lambda i: (0, i, 0))
_w_spec = pl.BlockSpec((D, D), lambda i: (0, 0))
_b_spec = pl.BlockSpec((1, D), lambda i: (0, 0))

_GRID = (N_NODES // B,)

_tc_h1 = pl.pallas_call(
    _tc_h1_body,
    grid=_GRID,
    in_specs=[_degT_spec, _row_spec, _w_spec],
    out_specs=_row_spec,
    out_shape=jax.ShapeDtypeStruct((N_NODES, D), jnp.float32),
)

_tc_mid = pl.pallas_call(
    _tc_mid_body,
    grid=_GRID,
    in_specs=[_degT_spec, _S_spec, _row_spec, _b_spec, _w_spec],
    out_specs=[_row_spec, _row_spec],
    out_shape=[jax.ShapeDtypeStruct((N_NODES, D), jnp.float32),
               jax.ShapeDtypeStruct((N_NODES, D), jnp.float32)],
)

_tc_fin = pl.pallas_call(
    _tc_fin_body,
    grid=_GRID,
    in_specs=[_degT_spec, _S_spec, _row_spec, _b_spec, _row_spec],
    out_specs=pl.BlockSpec((B, 2 * D), lambda i: (i, 0)),
    out_shape=jax.ShapeDtypeStruct((N_NODES, 2 * D), jnp.float32),
)


def kernel(x, edge_index, W1, b1, W2, b2):
    ei = edge_index.astype(jnp.int32)
    srcp = jnp.concatenate(
        [ei[0].reshape(NW, EPT), jnp.zeros((NW, PADW), jnp.int32)],
        axis=1).reshape(-1)
    padtgt = N_NODES + (jnp.arange(PADW, dtype=jnp.int32)
                        % (NPADE - N_NODES))
    dstp = jnp.concatenate(
        [ei[1].reshape(NW, EPT), jnp.broadcast_to(padtgt, (NW, PADW))],
        axis=1).reshape(-1)
    zeros_n = jnp.zeros((RCH,), jnp.float32)
    ones_c = jnp.ones((CP,), jnp.float32)
    zrows = jnp.zeros((CP, D), jnp.float32)
    b1r = b1.reshape(1, D)
    b2r = b2.reshape(1, D)

    deg_parts = _sc_deg(dstp, zeros_n, ones_c)
    degT = deg_parts.reshape(NC, N_NODES).T

    h1p = _tc_h1(degT, x, W1)
    S1 = _sc_edge(h1p, srcp, dstp, zrows).reshape(NC, N_NODES, D)
    h1o, h2p = _tc_mid(degT, S1, h1p, b1r, W2)
    S2 = _sc_edge(h2p, srcp, dstp, zrows).reshape(NC, N_NODES, D)
    out = _tc_fin(degT, S2, h2p, b2r, h1o)
    return out


# speedup vs baseline: 16.7599x; 1.0035x over previous
import functools

import jax
import jax.numpy as jnp
from jax import lax
from jax.experimental import pallas as pl
from jax.experimental.pallas import tpu as pltpu
from jax.experimental.pallas import tpu_sc as plsc

N_NODES = 10000
N_EDGES = 320000
D = 128

NC = 2
NS = 16
NW = NC * NS
EPT = N_EDGES // NW
CP = 128
NCHP = -(-EPT // CP)
EPTP = CP * NCHP
PADW = EPTP - EPT
NPADE = N_NODES + 16
RPT = 640
NPAD = NS * RPT
RCH = 80
FULL_CH = RPT // RCH
LAST_CH = (N_NODES - (NS - 1) * RPT) // RCH

NB = 3
NGRP = NCHP // NB
NTAIL = NCHP - NGRP * NB
NBD = 4
NGRPD = NCHP // NBD
NTAILD = NCHP - NGRPD * NBD
NFR = N_NODES // CP
LASTR = N_NODES - NFR * CP
NRSLOT = -(-(NFR + 1) // NS)

_mesh = plsc.VectorSubcoreMesh(core_axis_name="c", subcore_axis_name="s")


@functools.partial(
    pl.kernel,
    out_type=jax.ShapeDtypeStruct((NC * N_NODES,), jnp.float32),
    mesh=_mesh,
    scratch_types=(
        [pltpu.VMEM((CP,), jnp.float32),
         pltpu.VMEM((RCH,), jnp.float32)]
        + [pltpu.VMEM((CP,), jnp.int32)] * NBD
        + [pltpu.VMEM_SHARED((NPAD,), jnp.float32)]
        + [pltpu.SemaphoreType.DMA] * (2 * NBD)
    ),
)
def _sc_deg(dst_hbm, z80_hbm, ones_hbm, out_hbm, ones_v, db_v, *rest):
    didx = rest[:NBD]
    deg_sh = rest[NBD]
    xs = rest[NBD + 1:NBD + 1 + NBD]
    ss = rest[NBD + 1 + NBD:]
    c = lax.axis_index("c")
    s = lax.axis_index("s")
    wid = s * NC + c
    pltpu.sync_copy(z80_hbm, db_v)

    def initk(k, carry):
        @pl.when((s < NS - 1) | (k < LAST_CH))
        def _():
            pltpu.sync_copy(db_v, deg_sh.at[pl.ds(s * RPT + k * RCH, RCH)])
        return carry

    lax.fori_loop(0, FULL_CH, initk, 0)
    pltpu.sync_copy(ones_hbm, ones_v)
    for b in range(NBD):
        pltpu.async_copy(dst_hbm.at[pl.ds(wid * EPTP + b * CP, CP)], didx[b],
                         xs[b])
    plsc.subcore_barrier()

    def group(g, carry):
        for b in range(NBD):
            ci = g * NBD + b

            @pl.when(g > 0)
            def _():
                pltpu.make_async_copy(ones_hbm, ones_v, ss[b]).wait()
                pltpu.async_copy(dst_hbm.at[pl.ds(wid * EPTP + ci * CP, CP)],
                                 didx[b], xs[b])
        for b in range(NBD):
            pltpu.make_async_copy(dst_hbm.at[pl.ds(0, CP)], didx[b],
                                  xs[b]).wait()
            pltpu.async_copy(ones_v, deg_sh.at[didx[b]], ss[b], add=True)
        return carry

    lax.fori_loop(0, NGRPD, group, 0)
    for b in range(NBD):
        pltpu.make_async_copy(ones_hbm, ones_v, ss[b]).wait()
    for t in range(NTAILD):
        ci = NGRPD * NBD + t
        pltpu.sync_copy(dst_hbm.at[pl.ds(wid * EPTP + ci * CP, CP)], didx[t])
        pltpu.sync_copy(ones_v, deg_sh.at[didx[t]], add=True)
    plsc.subcore_barrier()

    def draink(k, carry):
        @pl.when((s < NS - 1) | (k < LAST_CH))
        def _():
            pltpu.sync_copy(deg_sh.at[pl.ds(s * RPT + k * RCH, RCH)], db_v)
            pltpu.sync_copy(
                db_v, out_hbm.at[pl.ds(c * N_NODES + s * RPT + k * RCH, RCH)])
        return carry

    lax.fori_loop(0, FULL_CH, draink, 0)


_edge_scratch = (
    [pltpu.VMEM_SHARED((NPADE, D), jnp.float32)]
    + [pltpu.VMEM((CP, D), jnp.float32)] * NB
    + [pltpu.VMEM((CP,), jnp.int32)] * NB
    + [pltpu.VMEM((CP,), jnp.int32)] * NB
    + [pltpu.SemaphoreType.DMA] * (4 * NB)
)


@functools.partial(
    pl.kernel,
    out_type=jax.ShapeDtypeStruct((NC * N_NODES, D), jnp.float32),
    mesh=_mesh,
    scratch_types=_edge_scratch,
)
def _sc_edge(h_hbm, src_hbm, dst_hbm, zrows_hbm, out_hbm, acc_sh, *ring):
    rows = ring[:NB]
    sidx = ring[NB:2 * NB]
    didx = ring[2 * NB:3 * NB]
    gs = ring[3 * NB:4 * NB]
    ss = ring[4 * NB:5 * NB]
    xs = ring[5 * NB:6 * NB]
    ds_ = ring[6 * NB:]
    zb_v = rows[0]
    c = lax.axis_index("c")
    s = lax.axis_index("s")
    wid = s * NC + c
    pltpu.sync_copy(zrows_hbm, zb_v)

    def initk(k, carry):
        j = k * NS + s

        @pl.when(j < NFR)
        def _():
            pltpu.sync_copy(zb_v, acc_sh.at[pl.ds(j * CP, CP)])

        @pl.when(j == NFR)
        def _():
            pltpu.sync_copy(zb_v.at[pl.ds(0, LASTR)],
                            acc_sh.at[pl.ds(NFR * CP, LASTR)])
        return carry

    lax.fori_loop(0, NRSLOT, initk, 0)
    for b in range(NB):
        pltpu.async_copy(src_hbm.at[pl.ds(wid * EPTP + b * CP, CP)], sidx[b],
                         xs[b])
        pltpu.async_copy(dst_hbm.at[pl.ds(wid * EPTP + b * CP, CP)], didx[b],
                         ds_[b])
    plsc.subcore_barrier()

    def group(g, carry):
        for b in range(NB):
            ci = g * NB + b

            @pl.when(g > 0)
            def _():
                pltpu.make_async_copy(
                    h_hbm.at[pl.ds(0, CP)], rows[b], ss[b]).wait()
                pltpu.async_copy(dst_hbm.at[pl.ds(wid * EPTP + ci * CP, CP)],
                                 didx[b], ds_[b])
            pltpu.make_async_copy(src_hbm.at[pl.ds(0, CP)], sidx[b],
                                  xs[b]).wait()
            pltpu.async_copy(h_hbm.at[sidx[b]], rows[b], gs[b])
        for b in range(NB):
            ci = g * NB + b
            pltpu.make_async_copy(h_hbm.at[pl.ds(0, CP)], rows[b],
                                  gs[b]).wait()

            @pl.when(ci + NB < NCHP)
            def _():
                pltpu.async_copy(
                    src_hbm.at[pl.ds(wid * EPTP + (ci + NB) * CP, CP)],
                    sidx[b], xs[b])
            pltpu.make_async_copy(dst_hbm.at[pl.ds(0, CP)], didx[b],
                                  ds_[b]).wait()
            pltpu.async_copy(rows[b], acc_sh.at[didx[b]], ss[b], add=True)
        return carry

    lax.fori_loop(0, NGRP, group, 0)
    for b in range(NB):
        pltpu.make_async_copy(h_hbm.at[pl.ds(0, CP)], rows[b], ss[b]).wait()
    for t in range(NTAIL):
        ci = NGRP * NB + t
        pltpu.make_async_copy(src_hbm.at[pl.ds(0, CP)], sidx[t], xs[t]).wait()
        pltpu.sync_copy(dst_hbm.at[pl.ds(wid * EPTP + ci * CP, CP)], didx[t])
        pltpu.async_copy(h_hbm.at[sidx[t]], rows[t], gs[t])
        pltpu.make_async_copy(h_hbm.at[pl.ds(0, CP)], rows[t], gs[t]).wait()
        pltpu.sync_copy(rows[t], acc_sh.at[didx[t]], add=True)
    plsc.subcore_barrier()

    def draink(k, carry):
        j = k * NS + s

        @pl.when(j < NFR)
        def _():
            pltpu.sync_copy(acc_sh.at[pl.ds(j * CP, CP)], zb_v)
            pltpu.sync_copy(zb_v,
                            out_hbm.at[pl.ds(c * N_NODES + j * CP, CP)])

        @pl.when(j == NFR)
        def _():
            pltpu.sync_copy(acc_sh.at[pl.ds(NFR * CP, LASTR)],
                            rows[1].at[pl.ds(0, LASTR)])
            pltpu.sync_copy(
                rows[1].at[pl.ds(0, LASTR)],
                out_hbm.at[pl.ds(c * N_NODES + NFR * CP, LASTR)])
        return carry

    lax.fori_loop(0, NRSLOT, draink, 0)


B = 1000


def _dinv_of(degT_blk):
    deg = degT_blk[:, 0] + degT_blk[:, 1] + 1.0
    return lax.rsqrt(deg)


def _tc_mm_body(x_ref, w_ref, h_ref):
    h_ref[...] = jnp.dot(x_ref[...], w_ref[...],
                         preferred_element_type=jnp.float32)


def _tc_scale_body(degT_ref, h_ref, h1p_ref):
    dinv = _dinv_of(degT_ref[...])
    h1p_ref[...] = h_ref[...] * dinv[:, None]


def _tc_mid_body(degT_ref, s_ref, h1p_ref, b1_ref, w2_ref, h1o_ref, h2p_ref):
    dinv = _dinv_of(degT_ref[...])
    agg = s_ref[0] + s_ref[1] + h1p_ref[...]
    h1o = jnp.maximum(dinv[:, None] * agg + b1_ref[...], 0.0)
    h1o_ref[...] = h1o
    h2 = jnp.dot(h1o, w2_ref[...], preferred_element_type=jnp.float32)
    h2p_ref[...] = h2 * dinv[:, None]


def _tc_fin_body(degT_ref, s_ref, h2p_ref, b2_ref, h1o_ref, out_ref):
    dinv = _dinv_of(degT_ref[...])
    agg = s_ref[0] + s_ref[1] + h2p_ref[...]
    h2o = jnp.maximum(dinv[:, None] * agg + b2_ref[...], 0.0)
    out_ref[:, :D] = h1o_ref[...]
    out_ref[:, D:] = h2o


_degT_spec = pl.BlockSpec((B, NC), lambda i: (i, 0))
_row_spec = pl.BlockSpec((B, D), lambda i: (i, 0))
_S_spec = pl.BlockSpec((NC, B, D), lambda i: (0, i, 0))
_w_spec = pl.BlockSpec((D, D), lambda i: (0, 0))
_b_spec = pl.BlockSpec((1, D), lambda i: (0, 0))

_GRID = (N_NODES // B,)

_tc_mm = pl.pallas_call(
    _tc_mm_body,
    grid=_GRID,
    in_specs=[_row_spec, _w_spec],
    out_specs=_row_spec,
    out_shape=jax.ShapeDtypeStruct((N_NODES, D), jnp.float32),
)

_tc_scale = pl.pallas_call(
    _tc_scale_body,
    grid=_GRID,
    in_specs=[_degT_spec, _row_spec],
    out_specs=_row_spec,
    out_shape=jax.ShapeDtypeStruct((N_NODES, D), jnp.float32),
)

_tc_mid = pl.pallas_call(
    _tc_mid_body,
    grid=_GRID,
    in_specs=[_degT_spec, _S_spec, _row_spec, _b_spec, _w_spec],
    out_specs=[_row_spec, _row_spec],
    out_shape=[jax.ShapeDtypeStruct((N_NODES, D), jnp.float32),
               jax.ShapeDtypeStruct((N_NODES, D), jnp.float32)],
)

_tc_fin = pl.pallas_call(
    _tc_fin_body,
    grid=_GRID,
    in_specs=[_degT_spec, _S_spec, _row_spec, _b_spec, _row_spec],
    out_specs=pl.BlockSpec((B, 2 * D), lambda i: (i, 0)),
    out_shape=jax.ShapeDtypeStruct((N_NODES, 2 * D), jnp.float32),
)


def kernel(x, edge_index, W1, b1, W2, b2):
    ei = edge_index.astype(jnp.int32)
    srcp = jnp.concatenate(
        [ei[0].reshape(NW, EPT), jnp.zeros((NW, PADW), jnp.int32)],
        axis=1).reshape(-1)
    padtgt = N_NODES + (jnp.arange(PADW, dtype=jnp.int32)
                        % (NPADE - N_NODES))
    dstp = jnp.concatenate(
        [ei[1].reshape(NW, EPT), jnp.broadcast_to(padtgt, (NW, PADW))],
        axis=1).reshape(-1)
    zeros_n = jnp.zeros((RCH,), jnp.float32)
    ones_c = jnp.ones((CP,), jnp.float32)
    zrows = jnp.zeros((CP, D), jnp.float32)
    b1r = b1.reshape(1, D)
    b2r = b2.reshape(1, D)

    h1 = _tc_mm(x, W1)
    deg_parts = _sc_deg(dstp, zeros_n, ones_c)
    degT = deg_parts.reshape(NC, N_NODES).T

    h1p = _tc_scale(degT, h1)
    S1 = _sc_edge(h1p, srcp, dstp, zrows).reshape(NC, N_NODES, D)
    h1o, h2p = _tc_mid(degT, S1, h1p, b1r, W2)
    S2 = _sc_edge(h2p, srcp, dstp, zrows).reshape(NC, N_NODES, D)
    out = _tc_fin(degT, S2, h2p, b2r, h1o)
    return out
